# baseline (speedup 1.0000x reference)
import functools

import jax
import jax.numpy as jnp
from jax import lax
from jax.experimental import pallas as pl
from jax.experimental.pallas import tpu as pltpu
from jax.experimental.pallas import tpu_sc as plsc

NUM_EMB = 1000000
D = 64
B = 4096
BAG = 50

NC = 2
NS = 16
NW = NC * NS

WT_COLS = NUM_EMB
BLK = 128
FULL_BLOCKS = WT_COLS // BLK
BLOCKS_PER_W = FULL_BLOCKS // NW
EXTRA_BLOCKS = FULL_BLOCKS - BLOCKS_PER_W * NW

BAGS_PER_W = B // NW
IDX_PER_W = BAGS_PER_W * BAG
COL_PER_W = BAGS_PER_W * D
CB = 8
CHUNK_IDX = CB * BAG
N_CHUNKS = BAGS_PER_W // CB
GATHER = 80
N_GATHER = CHUNK_IDX // GATHER


def _repack_block(blk, rep, iotas, npairs):
    def pair_body(p, carry):
        for h in range(2):
            col = jnp.full((16,), 0, jnp.int32) + (2 * p + h)
            for k in range(4):
                v = plsc.load_gather(blk, [iotas[k], col])
                rep[p, pl.ds(64 * h + 16 * k, 16)] = v
        return carry

    lax.fori_loop(0, npairs, pair_body, 0)


def _repack_body(wt, wtp, w2, blk0, blk1, rep0, rep1, si0, si1, so0, so1):
    wid = lax.axis_index("s") * NC + lax.axis_index("c")
    start = wid * BLOCKS_PER_W
    iotas = [lax.iota(jnp.int32, 16) + 16 * k for k in range(4)]

    def fire_in(buf, sem, bi):
        pltpu.async_copy(wt.at[:, pl.ds(bi * BLK, BLK)], buf, sem)

    def wait_in(buf, sem):
        pltpu.make_async_copy(wt.at[:, pl.ds(0, BLK)], buf, sem).wait()

    def fire_out(buf, sem, bi):
        pltpu.async_copy(buf, w2.at[pl.ds(bi * 64, 64)], sem)

    def wait_out(buf, sem):
        pltpu.make_async_copy(buf, w2.at[pl.ds(0, 64)], sem).wait()

    fire_in(blk0, si0, start)
    fire_in(blk1, si1, start + 1)

    def loop_body(u, carry):
        for j, blkj, repj, sij, soj in ((0, blk0, rep0, si0, so0),
                                        (1, blk1, rep1, si1, so1)):
            t = 2 * u + j
            bi = start + t
            wait_in(blkj, sij)

            @pl.when(t >= 2)
            def _():
                wait_out(repj, soj)

            _repack_block(blkj, repj, iotas, 64)
            fire_out(repj, soj, bi)

            @pl.when(t + 2 < BLOCKS_PER_W)
            def _():
                fire_in(blkj, sij, bi + 2)

        return carry

    lax.fori_loop(0, BLOCKS_PER_W // 2, loop_body, 0)
    wait_out(rep0, so0)
    wait_out(rep1, so1)

    @pl.when(wid < EXTRA_BLOCKS)
    def _():
        bi = NW * BLOCKS_PER_W + wid
        pltpu.sync_copy(wt.at[:, pl.ds(bi * BLK, BLK)], blk0)
        _repack_block(blk0, rep0, iotas, 64)
        pltpu.sync_copy(rep0, w2.at[pl.ds(bi * 64, 64)])

    @pl.when(wid == EXTRA_BLOCKS)
    def _():
        pltpu.sync_copy(wtp, blk0)
        _repack_block(blk0, rep0, iotas, 32)
        pltpu.sync_copy(rep0.at[pl.ds(0, 32)],
                        w2.at[pl.ds(FULL_BLOCKS * 64, 32)])


@functools.partial(
    pl.kernel,
    mesh=plsc.VectorSubcoreMesh(core_axis_name="c", subcore_axis_name="s"),
    out_type=jax.ShapeDtypeStruct((NUM_EMB // 2, 2 * D), jnp.float32),
    compiler_params=pltpu.CompilerParams(needs_layout_passes=False),
    scratch_types=[
        pltpu.VMEM((D, BLK), jnp.float32),
        pltpu.VMEM((D, BLK), jnp.float32),
        pltpu.VMEM((64, 2 * D), jnp.float32),
        pltpu.VMEM((64, 2 * D), jnp.float32),
        pltpu.SemaphoreType.DMA,
        pltpu.SemaphoreType.DMA,
        pltpu.SemaphoreType.DMA,
        pltpu.SemaphoreType.DMA,
    ],
)
def _repack(wt, wtp, w2, blk0, blk1, rep0, rep1, si0, si1, so0, so1):
    _repack_body(wt, wtp, w2, blk0, blk1, rep0, rep1, si0, si1, so0, so1)


def _ebag_body(idx_hbm, col_hbm, table_hbm, out_hbm, idx_v, col_v, rows_v, out_v, sem):
    wid = lax.axis_index("s") * NC + lax.axis_index("c")
    pltpu.sync_copy(idx_hbm.at[pl.ds(wid * IDX_PER_W, IDX_PER_W)], idx_v)
    pltpu.sync_copy(col_hbm.at[pl.ds(wid * COL_PER_W, COL_PER_W)], col_v)

    def chunk_body(c, carry):
        base = c * CHUNK_IDX
        copies = [
            pltpu.async_copy(
                table_hbm.at[idx_v.at[pl.ds(base + j * GATHER, GATHER)]],
                rows_v.at[pl.ds(j * GATHER, GATHER)],
                sem,
            )
            for j in range(N_GATHER)
        ]
        for cp in copies:
            cp.wait()

        def bag_body(b, carry2):
            row0 = b * BAG
            g = c * CB + b
            accs = [jnp.zeros((16,), jnp.float32) for _ in range(4)]
            for t in range(4):
                cv = col_v[pl.ds(g * D + t * 16, 16)]
                for i in range(16 if t < 3 else BAG - 48):
                    r = t * 16 + i
                    col = cv[i]
                    for k in range(4):
                        accs[k] = accs[k] + rows_v[row0 + r, pl.ds(col + k * 16, 16)]
            orow = g >> 1
            ocol = (g & 1) * D
            for k in range(4):
                out_v[orow, pl.ds(ocol + k * 16, 16)] = accs[k] * jnp.float32(1.0 / BAG)
            return carry2

        lax.fori_loop(0, CB, bag_body, 0)
        return carry

    lax.fori_loop(0, N_CHUNKS, chunk_body, 0)
    pltpu.sync_copy(out_v, out_hbm.at[pl.ds(wid * (BAGS_PER_W // 2), BAGS_PER_W // 2)])


@functools.partial(
    pl.kernel,
    mesh=plsc.VectorSubcoreMesh(core_axis_name="c", subcore_axis_name="s"),
    out_type=jax.ShapeDtypeStruct((B // 2, 2 * D), jnp.float32),
    scratch_types=[
        pltpu.VMEM((IDX_PER_W,), jnp.int32),
        pltpu.VMEM((COL_PER_W,), jnp.int32),
        pltpu.VMEM((CHUNK_IDX, 2 * D), jnp.float32),
        pltpu.VMEM((BAGS_PER_W // 2, 2 * D), jnp.float32),
        pltpu.SemaphoreType.DMA,
    ],
)
def _ebag(idx_hbm, col_hbm, table_hbm, out_hbm, idx_v, col_v, rows_v, out_v, sem):
    _ebag_body(idx_hbm, col_hbm, table_hbm, out_hbm, idx_v, col_v, rows_v, out_v, sem)


def kernel(input, weight):
    idx = jnp.asarray(input, jnp.int32)
    pair = (idx >> 1).reshape(-1)
    col = jnp.pad((idx & 1) * D, ((0, 0), (0, D - BAG))).reshape(-1)
    wt = weight.T
    wtp = jnp.pad(weight[FULL_BLOCKS * BLK :].T, ((0, 0), (0, D)))
    w2 = _repack(wt, wtp)
    out2 = _ebag(pair, col, w2)
    return out2.reshape(B, D)


# speedup vs baseline: 2.4738x; 2.4738x over previous
import functools

import jax
import jax.numpy as jnp
from jax import lax
from jax.experimental import pallas as pl
from jax.experimental.pallas import tpu as pltpu
from jax.experimental.pallas import tpu_sc as plsc

NUM_EMB = 1000000
D = 64
B = 4096
BAG = 50

NC = 2
NS = 16
NW = NC * NS

BAGS_PER_W = B // NW
IDX_PER_W = BAGS_PER_W * BAG
CB = 8
CHUNK_IDX = CB * BAG
N_CHUNKS = BAGS_PER_W // CB
GATHER = 80
N_GATHER = CHUNK_IDX // GATHER


def _ebag_body(idx_hbm, table_hbm, out_hbm, idx_v, rows0, rows1, out_v, sem0, sem1):
    wid = lax.axis_index("s") * NC + lax.axis_index("c")
    pltpu.sync_copy(idx_hbm.at[pl.ds(wid * IDX_PER_W, IDX_PER_W)], idx_v)

    bufs = (rows0, rows1)
    sems = (sem0, sem1)

    def fire(c, buf, sem):
        base = c * CHUNK_IDX
        for j in range(N_GATHER):
            pltpu.async_copy(
                table_hbm.at[idx_v.at[pl.ds(base + j * GATHER, GATHER)]],
                buf.at[pl.ds(j * GATHER, GATHER)],
                sem,
            )

    def drain(buf, sem):
        for j in range(N_GATHER):
            pltpu.make_async_copy(
                table_hbm.at[idx_v.at[pl.ds(j * GATHER, GATHER)]],
                buf.at[pl.ds(j * GATHER, GATHER)],
                sem,
            ).wait()

    def accumulate(c, buf):
        def bag_body(b, carry):
            row0 = b * BAG
            accs = [jnp.zeros((16,), jnp.float32) for _ in range(4)]
            for r in range(BAG):
                for k in range(4):
                    accs[k] = accs[k] + buf[row0 + r, pl.ds(k * 16, 16)]
            for k in range(4):
                out_v[c * CB + b, pl.ds(k * 16, 16)] = accs[k] * jnp.float32(1.0 / BAG)
            return carry

        lax.fori_loop(0, CB, bag_body, 0)

    fire(0, rows0, sem0)
    fire(1, rows1, sem1)

    def chunk_body(u, carry):
        for j in range(2):
            c = 2 * u + j
            drain(bufs[j], sems[j])
            accumulate(c, bufs[j])

            @pl.when(c + 2 < N_CHUNKS)
            def _():
                fire(c + 2, bufs[j], sems[j])

        return carry

    lax.fori_loop(0, N_CHUNKS // 2, chunk_body, 0)
    pltpu.sync_copy(out_v, out_hbm.at[pl.ds(wid * BAGS_PER_W, BAGS_PER_W)])


@functools.partial(
    pl.kernel,
    mesh=plsc.VectorSubcoreMesh(core_axis_name="c", subcore_axis_name="s"),
    out_type=jax.ShapeDtypeStruct((B, D), jnp.float32),
    compiler_params=pltpu.CompilerParams(use_tc_tiling_on_sc=False),
    scratch_types=[
        pltpu.VMEM((IDX_PER_W,), jnp.int32),
        pltpu.VMEM((CHUNK_IDX, D), jnp.float32),
        pltpu.VMEM((CHUNK_IDX, D), jnp.float32),
        pltpu.VMEM((BAGS_PER_W, D), jnp.float32),
        pltpu.SemaphoreType.DMA,
        pltpu.SemaphoreType.DMA,
    ],
)
def _ebag(idx_hbm, table_hbm, out_hbm, idx_v, rows0, rows1, out_v, sem0, sem1):
    _ebag_body(idx_hbm, table_hbm, out_hbm, idx_v, rows0, rows1, out_v, sem0, sem1)


def kernel(input, weight):
    idx = jnp.asarray(input, jnp.int32).reshape(-1)
    return _ebag(idx, weight)


# speedup vs baseline: 2.4752x; 1.0006x over previous
import functools

import jax
import jax.numpy as jnp
from jax import lax
from jax.experimental import pallas as pl
from jax.experimental.pallas import tpu as pltpu
from jax.experimental.pallas import tpu_sc as plsc

NUM_EMB = 1000000
D = 64
B = 4096
BAG = 50

NC = 2
NS = 16
NW = NC * NS

BAGS_PER_W = B // NW
IDX_PER_W = BAGS_PER_W * BAG
CB = 16
CHUNK_IDX = CB * BAG
N_CHUNKS = BAGS_PER_W // CB
GATHER = 80
N_GATHER = CHUNK_IDX // GATHER


def _ebag_body(idx_hbm, table_hbm, out_hbm, idx_v, rows0, rows1, out_v, sem0, sem1):
    wid = lax.axis_index("s") * NC + lax.axis_index("c")
    pltpu.sync_copy(idx_hbm.at[pl.ds(wid * IDX_PER_W, IDX_PER_W)], idx_v)

    bufs = (rows0, rows1)
    sems = (sem0, sem1)

    def fire(c, buf, sem):
        base = c * CHUNK_IDX
        for j in range(N_GATHER):
            pltpu.async_copy(
                table_hbm.at[idx_v.at[pl.ds(base + j * GATHER, GATHER)]],
                buf.at[pl.ds(j * GATHER, GATHER)],
                sem,
            )

    def drain(buf, sem):
        for j in range(N_GATHER):
            pltpu.make_async_copy(
                table_hbm.at[idx_v.at[pl.ds(j * GATHER, GATHER)]],
                buf.at[pl.ds(j * GATHER, GATHER)],
                sem,
            ).wait()

    def accumulate(c, buf):
        def bag_body(b, carry):
            row0 = b * BAG
            accs = [jnp.zeros((16,), jnp.float32) for _ in range(4)]
            for r in range(BAG):
                for k in range(4):
                    accs[k] = accs[k] + buf[row0 + r, pl.ds(k * 16, 16)]
            for k in range(4):
                out_v[c * CB + b, pl.ds(k * 16, 16)] = accs[k] * jnp.float32(1.0 / BAG)
            return carry

        lax.fori_loop(0, CB, bag_body, 0)

    fire(0, rows0, sem0)
    fire(1, rows1, sem1)

    def chunk_body(u, carry):
        for j in range(2):
            c = 2 * u + j
            drain(bufs[j], sems[j])
            accumulate(c, bufs[j])

            @pl.when(c + 2 < N_CHUNKS)
            def _():
                fire(c + 2, bufs[j], sems[j])

        return carry

    lax.fori_loop(0, N_CHUNKS // 2, chunk_body, 0)
    pltpu.sync_copy(out_v, out_hbm.at[pl.ds(wid * BAGS_PER_W, BAGS_PER_W)])


@functools.partial(
    pl.kernel,
    mesh=plsc.VectorSubcoreMesh(core_axis_name="c", subcore_axis_name="s"),
    out_type=jax.ShapeDtypeStruct((B, D), jnp.float32),
    compiler_params=pltpu.CompilerParams(use_tc_tiling_on_sc=False),
    scratch_types=[
        pltpu.VMEM((IDX_PER_W,), jnp.int32),
        pltpu.VMEM((CHUNK_IDX, D), jnp.float32),
        pltpu.VMEM((CHUNK_IDX, D), jnp.float32),
        pltpu.VMEM((BAGS_PER_W, D), jnp.float32),
        pltpu.SemaphoreType.DMA,
        pltpu.SemaphoreType.DMA,
    ],
)
def _ebag(idx_hbm, table_hbm, out_hbm, idx_v, rows0, rows1, out_v, sem0, sem1):
    _ebag_body(idx_hbm, table_hbm, out_hbm, idx_v, rows0, rows1, out_v, sem0, sem1)


def kernel(input, weight):
    idx = jnp.asarray(input, jnp.int32).reshape(-1)
    return _ebag(idx, weight)
